# baseline (speedup 1.0000x reference)
import functools
import jax
import jax.numpy as jnp
from jax import lax
from jax.experimental import pallas as pl
from jax.experimental.pallas import tpu as pltpu
from jax.experimental.pallas import tpu_sc as plsc

N_NODES = 10000
N_SUB = 16
EDGE_VALID = 10000
EDGE_SLOTS = 10240
N_ACC = 10240
ROWS_PER_SUB = N_ACC // 16


def _lrelu(x, slope):
    return jnp.where(x >= 0, x, slope * x)


def _pad_p(dout):
    h = ((dout // 2) // 16 + 1) * 16
    return 2 * h


def _dense_body(x_ref, w_ref, as_ref, ad_ref, l_ref, h_ref, aa_ref, r_ref,
                cv_ref, c11_ref):
    x = x_ref[...]
    h = jnp.dot(x, w_ref[...], preferred_element_type=jnp.float32)
    h_ref[...] = h
    asrc = jnp.dot(h, as_ref[...], preferred_element_type=jnp.float32)
    adst = jnp.dot(h, ad_ref[...], preferred_element_type=jnp.float32)
    aa_ref[...] = jnp.concatenate([asrc, adst], axis=1)
    if l_ref is not None:
        r_ref[...] = jnp.dot(x, l_ref[...], preferred_element_type=jnp.float32)
    c = jnp.max(asrc) + jnp.max(adst)
    c = jnp.where(c >= 0, c, 0.2 * c)
    cv_ref[...] = jnp.full((16,), c, jnp.float32)
    c11_ref[...] = c.reshape(1, 1)


def _dense_layer(x, Wp, a_sp, a_dp, L):
    N = x.shape[0]
    P = Wp.shape[1]
    has_l = L is not None
    outs = [
        jax.ShapeDtypeStruct((N, P), jnp.float32),
        jax.ShapeDtypeStruct((N, 2), jnp.float32),
        jax.ShapeDtypeStruct((N, L.shape[1] if has_l else 1), jnp.float32),
        jax.ShapeDtypeStruct((16,), jnp.float32),
        jax.ShapeDtypeStruct((1, 1), jnp.float32),
    ]
    body = _dense_body if has_l else (
        lambda x_ref, w_ref, as_ref, ad_ref, h_ref, aa_ref, r_ref, cv_ref,
        c11_ref: _dense_body(x_ref, w_ref, as_ref, ad_ref, None, h_ref,
                             aa_ref, r_ref, cv_ref, c11_ref))
    args = (x, Wp, a_sp, a_dp) + ((L,) if has_l else ())
    return pl.pallas_call(body, out_shape=outs)(*args)


@functools.partial(jax.jit, static_argnames=("H", "dout", "BLK"))
def _sc_edge(h2, asrc, adst, cvec, src3, dst3, *, H, dout, BLK):
    mesh = plsc.VectorSubcoreMesh(core_axis_name="c", subcore_axis_name="s",
                                  num_cores=2)
    nvec = H // 16
    NBLK = EDGE_SLOTS // BLK

    @functools.partial(
        pl.kernel,
        out_type=jax.ShapeDtypeStruct((2, N_ACC, H), jnp.float32),
        mesh=mesh,
        scratch_types=[
            pltpu.VMEM((N_NODES,), jnp.float32),
            pltpu.VMEM((N_NODES,), jnp.float32),
            pltpu.VMEM((16,), jnp.float32),
            pltpu.VMEM((NBLK + 1, BLK), jnp.int32),
            pltpu.VMEM((NBLK + 1, BLK), jnp.int32),
            pltpu.VMEM((BLK,), jnp.float32),
            pltpu.VMEM((BLK,), jnp.float32),
            pltpu.VMEM((BLK, H), jnp.float32),
            pltpu.VMEM((BLK, H), jnp.float32),
            pltpu.VMEM_SHARED((N_ACC, H), jnp.float32),
            pltpu.SemaphoreType.DMA,
            pltpu.SemaphoreType.DMA,
        ],
        compiler_params=pltpu.CompilerParams(needs_layout_passes=False,
                                             use_tc_tiling_on_sc=False),
    )
    def k(h_hbm, asrc_hbm, adst_hbm, c_hbm, src_hbm, dst_hbm, out_hbm,
          asrc_v, adst_v, cv_v, src_v, dst_v, ex_a, ex_b, rows_a, rows_b,
          acc_sh, sem0, sem1):
        c = lax.axis_index("c")
        s = lax.axis_index("s")
        pltpu.sync_copy(asrc_hbm, asrc_v)
        pltpu.sync_copy(adst_hbm, adst_v)
        pltpu.sync_copy(c_hbm, cv_v)
        pltpu.sync_copy(src_hbm.at[s], src_v)
        pltpu.sync_copy(dst_hbm.at[s], dst_v)

        def zbody(i, _):
            for j in range(nvec):
                rows_a[i, pl.ds(j * 16, 16)] = jnp.zeros((16,), jnp.float32)
            return 0
        lax.fori_loop(0, BLK, zbody, 0)
        base = s * ROWS_PER_SUB
        for t in range(ROWS_PER_SUB // BLK):
            pltpu.sync_copy(rows_a, acc_sh.at[pl.ds(base + t * BLK, BLK)])
        plsc.subcore_barrier()

        cv = cv_v[...]
        colbase = c * H

        def compute_ex(b, ex_v):
            for j in range(BLK // 16):
                s16 = src_v[b, pl.ds(j * 16, 16)]
                d16 = dst_v[b, pl.ds(j * 16, 16)]
                a1 = plsc.load_gather(asrc_v, [s16])
                a2 = plsc.load_gather(adst_v, [d16])
                e = a1 + a2
                e = jnp.where(e >= 0, e, 0.2 * e) - cv
                ex = jnp.exp(e)
                gid = b * BLK + j * 16 + lax.iota(jnp.int32, 16)
                ex = jnp.where(gid < EDGE_VALID, ex, 0.0)
                ex_v[pl.ds(j * 16, 16)] = ex

        def process(b, ex_v, rows_v, sem):
            compute_ex(b, ex_v)
            pltpu.async_copy(h_hbm.at[c].at[src_v.at[b]], rows_v, sem).wait()

            def sbody(i, _):
                exb = plsc.load_gather(ex_v, [jnp.full((16,), i, jnp.int32)])
                for j in range(nvec):
                    v = rows_v[i, pl.ds(j * 16, 16)] * exb
                    colids = colbase + j * 16 + lax.iota(jnp.int32, 16)
                    v = jnp.where(colids == dout, exb, v)
                    rows_v[i, pl.ds(j * 16, 16)] = v
                return 0
            lax.fori_loop(0, BLK, sbody, 0)
            pltpu.sync_copy(rows_v, acc_sh.at[dst_v.at[b]], add=True)

        def pair(k2, _):
            process(2 * k2, ex_a, rows_a, sem0)
            process(2 * k2 + 1, ex_b, rows_b, sem1)
            return 0
        lax.fori_loop(0, NBLK // 2, pair, 0)

        plsc.subcore_barrier()
        for t in range(ROWS_PER_SUB // BLK):
            pltpu.sync_copy(acc_sh.at[pl.ds(base + t * BLK, BLK)],
                            out_hbm.at[c, pl.ds(base + t * BLK, BLK)])

    return k(h2, asrc, adst, cvec, src3, dst3)


def _norm_layer(o0, o1, h, aa, c11, r, b2, lb2, dout):
    N, P = h.shape
    H = P // 2
    has_l = r is not None
    RB = 2000
    grid = (N // RB,)

    def body(*refs):
        if has_l:
            o0_ref, o1_ref, h_ref, aa_ref, c_ref, r_ref, b_ref, lb_ref, out_ref = refs
        else:
            o0_ref, o1_ref, h_ref, aa_ref, c_ref, b_ref, out_ref = refs
        es = jnp.exp(_lrelu(aa_ref[:, 0:1] + aa_ref[:, 1:2], 0.2) - c_ref[0, 0])
        acc = jnp.concatenate([o0_ref[...], o1_ref[...]], axis=1)
        acc = acc + es * h_ref[...]
        denom = acc[:, dout:dout + 1] + es
        g = acc[:, 0:dout] / denom + b_ref[...]
        if has_l:
            out_ref[...] = _lrelu(g, 0.01) + r_ref[...] + lb_ref[...]
        else:
            out_ref[...] = g

    rowspec = lambda w: pl.BlockSpec((RB, w), lambda i: (i, 0))
    fixspec = lambda a, b: pl.BlockSpec((a, b), lambda i: (0, 0))
    in_specs = [rowspec(H), rowspec(H), rowspec(P), rowspec(2), fixspec(1, 1)]
    args = [o0, o1, h, aa, c11]
    if has_l:
        in_specs += [rowspec(r.shape[1]), fixspec(1, dout), fixspec(1, dout)]
        args += [r, b2, lb2]
    else:
        in_specs += [fixspec(1, dout)]
        args += [b2]
    return pl.pallas_call(
        body, grid=grid, in_specs=in_specs,
        out_specs=rowspec(dout),
        out_shape=jax.ShapeDtypeStruct((N, dout), jnp.float32))(*args)


def kernel(x, edge_index,
           W1, as1, ad1, b1, L1, lb1,
           W2, as2, ad2, b2, L2, lb2,
           W3, as3, ad3, b3, L3, lb3,
           W4, as4, ad4, b4, L4, lb4,
           W5, as5, ad5, b5, L5, lb5,
           W6, as6, ad6, b6, L6, lb6,
           W7, as7, ad7, b7):
    src = edge_index[0].astype(jnp.int32)
    dst = edge_index[1].astype(jnp.int32)
    padw = EDGE_SLOTS - EDGE_VALID + 256
    padv = (jnp.arange(N_SUB * padw, dtype=jnp.int32) % N_NODES).reshape(
        N_SUB, padw)
    src2 = jnp.concatenate([src.reshape(N_SUB, EDGE_VALID), padv], axis=1)
    dst2 = jnp.concatenate([dst.reshape(N_SUB, EDGE_VALID), padv], axis=1)

    p = {
        "W1": W1, "as1": as1, "ad1": ad1, "b1": b1, "L1": L1, "lb1": lb1,
        "W2": W2, "as2": as2, "ad2": ad2, "b2": b2, "L2": L2, "lb2": lb2,
        "W3": W3, "as3": as3, "ad3": ad3, "b3": b3, "L3": L3, "lb3": lb3,
        "W4": W4, "as4": as4, "ad4": ad4, "b4": b4, "L4": L4, "lb4": lb4,
        "W5": W5, "as5": as5, "ad5": ad5, "b5": b5, "L5": L5, "lb5": lb5,
        "W6": W6, "as6": as6, "ad6": ad6, "b6": b6, "L6": L6, "lb6": lb6,
        "W7": W7, "as7": as7, "ad7": ad7, "b7": b7,
    }
    h = x
    for li in range(1, 8):
        W = p[f"W{li}"]
        dout = W.shape[1]
        P = _pad_p(dout)
        H = P // 2
        Wp = jnp.pad(W, ((0, 0), (0, P - dout)))
        a_sp = jnp.pad(p[f"as{li}"], (0, P - dout)).reshape(P, 1)
        a_dp = jnp.pad(p[f"ad{li}"], (0, P - dout)).reshape(P, 1)
        has_l = li < 7
        L = p[f"L{li}"] if has_l else None
        hW, aa, r, cvec, c11 = _dense_layer(h, Wp, a_sp, a_dp, L)
        h2 = jnp.stack([hW[:, :H], hW[:, H:]])
        blk = 64 if H > 64 else 128
        nblk = EDGE_SLOTS // blk
        cols = EDGE_SLOTS + blk
        src3 = src2[:, :cols].reshape(N_SUB, nblk + 1, blk)
        dst3 = dst2[:, :cols].reshape(N_SUB, nblk + 1, blk)
        o2 = _sc_edge(h2, aa[:, 0], aa[:, 1], cvec, src3, dst3,
                      H=H, dout=dout, BLK=blk)
        b2 = p[f"b{li}"].reshape(1, dout)
        if has_l:
            h = _norm_layer(o2[0], o2[1], hW, aa, c11, r, b2,
                            p[f"lb{li}"].reshape(1, dout), dout)
        else:
            return _norm_layer(o2[0], o2[1], hW, aa, c11, None, b2, None,
                               dout)


# speedup vs baseline: 1.4101x; 1.4101x over previous
import functools
import jax
import jax.numpy as jnp
from jax import lax
from jax.experimental import pallas as pl
from jax.experimental.pallas import tpu as pltpu
from jax.experimental.pallas import tpu_sc as plsc

N_NODES = 10000
N_SUB = 16
EDGE_VALID = 10000
EDGE_SLOTS = 10240
N_ACC = 10240
ROWS_PER_SUB = N_ACC // 16


def _lrelu(x, slope):
    return jnp.where(x >= 0, x, slope * x)


def _pad_p(dout):
    h = ((dout // 2) // 16 + 1) * 16
    return 2 * h


def _dense_body(x_ref, w_ref, as_ref, ad_ref, l_ref, h_ref, aa_ref, r_ref,
                cv_ref, c11_ref):
    x = x_ref[...]
    h = jnp.dot(x, w_ref[...], preferred_element_type=jnp.float32)
    h_ref[...] = h
    asrc = jnp.dot(h, as_ref[...], preferred_element_type=jnp.float32)
    adst = jnp.dot(h, ad_ref[...], preferred_element_type=jnp.float32)
    aa_ref[...] = jnp.concatenate([asrc, adst], axis=1)
    if l_ref is not None:
        r_ref[...] = jnp.dot(x, l_ref[...], preferred_element_type=jnp.float32)
    c = jnp.max(asrc) + jnp.max(adst)
    c = jnp.where(c >= 0, c, 0.2 * c)
    cv_ref[...] = jnp.full((16,), c, jnp.float32)
    c11_ref[...] = c.reshape(1, 1)


def _dense_layer(x, Wp, a_sp, a_dp, L):
    N = x.shape[0]
    P = Wp.shape[1]
    has_l = L is not None
    outs = [
        jax.ShapeDtypeStruct((N, P), jnp.float32),
        jax.ShapeDtypeStruct((N, 2), jnp.float32),
        jax.ShapeDtypeStruct((N, L.shape[1] if has_l else 1), jnp.float32),
        jax.ShapeDtypeStruct((16,), jnp.float32),
        jax.ShapeDtypeStruct((1, 1), jnp.float32),
    ]
    body = _dense_body if has_l else (
        lambda x_ref, w_ref, as_ref, ad_ref, h_ref, aa_ref, r_ref, cv_ref,
        c11_ref: _dense_body(x_ref, w_ref, as_ref, ad_ref, None, h_ref,
                             aa_ref, r_ref, cv_ref, c11_ref))
    args = (x, Wp, a_sp, a_dp) + ((L,) if has_l else ())
    return pl.pallas_call(body, out_shape=outs)(*args)


@functools.partial(jax.jit, static_argnames=("H", "dout", "BLK"))
def _sc_edge(h2, asrc, adst, cvec, src3, dst3, *, H, dout, BLK):
    mesh = plsc.VectorSubcoreMesh(core_axis_name="c", subcore_axis_name="s",
                                  num_cores=2)
    nvec = H // 16
    NBLK = EDGE_SLOTS // BLK

    @functools.partial(
        pl.kernel,
        out_type=jax.ShapeDtypeStruct((2, N_ACC, H), jnp.float32),
        mesh=mesh,
        scratch_types=[
            pltpu.VMEM((N_NODES,), jnp.float32),
            pltpu.VMEM((N_NODES,), jnp.float32),
            pltpu.VMEM((16,), jnp.float32),
            pltpu.VMEM((NBLK + 1, BLK), jnp.int32),
            pltpu.VMEM((NBLK + 1, BLK), jnp.int32),
            pltpu.VMEM((BLK,), jnp.float32),
            pltpu.VMEM((BLK,), jnp.float32),
            pltpu.VMEM((BLK, H), jnp.float32),
            pltpu.VMEM((BLK, H), jnp.float32),
            pltpu.VMEM_SHARED((N_ACC, H), jnp.float32),
            pltpu.SemaphoreType.DMA,
            pltpu.SemaphoreType.DMA,
        ],
        compiler_params=pltpu.CompilerParams(needs_layout_passes=False,
                                             use_tc_tiling_on_sc=False),
    )
    def k(h_hbm, asrc_hbm, adst_hbm, c_hbm, src_hbm, dst_hbm, out_hbm,
          asrc_v, adst_v, cv_v, src_v, dst_v, ex_a, ex_b, rows_a, rows_b,
          acc_sh, sem0, sem1):
        c = lax.axis_index("c")
        s = lax.axis_index("s")
        pltpu.sync_copy(asrc_hbm, asrc_v)
        pltpu.sync_copy(adst_hbm, adst_v)
        pltpu.sync_copy(c_hbm, cv_v)
        pltpu.sync_copy(src_hbm.at[s], src_v)
        pltpu.sync_copy(dst_hbm.at[s], dst_v)

        def zbody(i, _):
            for j in range(nvec):
                rows_a[i, pl.ds(j * 16, 16)] = jnp.zeros((16,), jnp.float32)
            return 0
        lax.fori_loop(0, BLK, zbody, 0)
        base = s * ROWS_PER_SUB
        for t in range(ROWS_PER_SUB // BLK):
            pltpu.sync_copy(rows_a, acc_sh.at[pl.ds(base + t * BLK, BLK)])
        plsc.subcore_barrier()

        cv = cv_v[...]
        colbase = c * H

        def compute_ex(b, ex_v):
            for j in range(BLK // 16):
                s16 = src_v[b, pl.ds(j * 16, 16)]
                d16 = dst_v[b, pl.ds(j * 16, 16)]
                a1 = plsc.load_gather(asrc_v, [s16])
                a2 = plsc.load_gather(adst_v, [d16])
                e = a1 + a2
                e = jnp.where(e >= 0, e, 0.2 * e) - cv
                ex = jnp.exp(e)
                gid = b * BLK + j * 16 + lax.iota(jnp.int32, 16)
                ex = jnp.where(gid < EDGE_VALID, ex, 0.0)
                ex_v[pl.ds(j * 16, 16)] = ex

        def start_gather(b, rows_v, sem):
            pltpu.async_copy(h_hbm.at[c].at[src_v.at[b]], rows_v, sem)

        def wait_gather(b, rows_v, sem):
            pltpu.make_async_copy(h_hbm.at[c].at[src_v.at[b]], rows_v,
                                  sem).wait()

        def process(b, ex_v, rows_v, sem, ex_n, rows_n, sem_n):
            compute_ex(b + 1, ex_n)
            start_gather(b + 1, rows_n, sem_n)
            wait_gather(b, rows_v, sem)

            def sbody(i, _):
                exb = plsc.load_gather(ex_v, [jnp.full((16,), i, jnp.int32)])
                for j in range(nvec):
                    v = rows_v[i, pl.ds(j * 16, 16)] * exb
                    colids = colbase + j * 16 + lax.iota(jnp.int32, 16)
                    v = jnp.where(colids == dout, exb, v)
                    rows_v[i, pl.ds(j * 16, 16)] = v
                return 0
            lax.fori_loop(0, BLK, sbody, 0)
            pltpu.sync_copy(rows_v, acc_sh.at[dst_v.at[b]], add=True)

        compute_ex(0, ex_a)
        start_gather(0, rows_a, sem0)

        def pair(k2, _):
            process(2 * k2, ex_a, rows_a, sem0, ex_b, rows_b, sem1)
            process(2 * k2 + 1, ex_b, rows_b, sem1, ex_a, rows_a, sem0)
            return 0
        lax.fori_loop(0, NBLK // 2, pair, 0)
        wait_gather(NBLK, rows_a, sem0)

        plsc.subcore_barrier()
        for t in range(ROWS_PER_SUB // BLK):
            pltpu.sync_copy(acc_sh.at[pl.ds(base + t * BLK, BLK)],
                            out_hbm.at[c, pl.ds(base + t * BLK, BLK)])

    return k(h2, asrc, adst, cvec, src3, dst3)


def _norm_layer(o0, o1, h, aa, c11, r, b2, lb2, dout):
    N, P = h.shape
    H = P // 2
    has_l = r is not None
    RB = 2000
    grid = (N // RB,)

    def body(*refs):
        if has_l:
            o0_ref, o1_ref, h_ref, aa_ref, c_ref, r_ref, b_ref, lb_ref, out_ref = refs
        else:
            o0_ref, o1_ref, h_ref, aa_ref, c_ref, b_ref, out_ref = refs
        es = jnp.exp(_lrelu(aa_ref[:, 0:1] + aa_ref[:, 1:2], 0.2) - c_ref[0, 0])
        acc = jnp.concatenate([o0_ref[...], o1_ref[...]], axis=1)
        acc = acc + es * h_ref[...]
        denom = acc[:, dout:dout + 1] + es
        g = acc[:, 0:dout] / denom + b_ref[...]
        if has_l:
            out_ref[...] = _lrelu(g, 0.01) + r_ref[...] + lb_ref[...]
        else:
            out_ref[...] = g

    rowspec = lambda w: pl.BlockSpec((RB, w), lambda i: (i, 0))
    fixspec = lambda a, b: pl.BlockSpec((a, b), lambda i: (0, 0))
    in_specs = [rowspec(H), rowspec(H), rowspec(P), rowspec(2), fixspec(1, 1)]
    args = [o0, o1, h, aa, c11]
    if has_l:
        in_specs += [rowspec(r.shape[1]), fixspec(1, dout), fixspec(1, dout)]
        args += [r, b2, lb2]
    else:
        in_specs += [fixspec(1, dout)]
        args += [b2]
    return pl.pallas_call(
        body, grid=grid, in_specs=in_specs,
        out_specs=rowspec(dout),
        out_shape=jax.ShapeDtypeStruct((N, dout), jnp.float32))(*args)


def kernel(x, edge_index,
           W1, as1, ad1, b1, L1, lb1,
           W2, as2, ad2, b2, L2, lb2,
           W3, as3, ad3, b3, L3, lb3,
           W4, as4, ad4, b4, L4, lb4,
           W5, as5, ad5, b5, L5, lb5,
           W6, as6, ad6, b6, L6, lb6,
           W7, as7, ad7, b7):
    src = edge_index[0].astype(jnp.int32)
    dst = edge_index[1].astype(jnp.int32)
    padw = EDGE_SLOTS - EDGE_VALID + 256
    padv = (jnp.arange(N_SUB * padw, dtype=jnp.int32) % N_NODES).reshape(
        N_SUB, padw)
    src2 = jnp.concatenate([src.reshape(N_SUB, EDGE_VALID), padv], axis=1)
    dst2 = jnp.concatenate([dst.reshape(N_SUB, EDGE_VALID), padv], axis=1)

    p = {
        "W1": W1, "as1": as1, "ad1": ad1, "b1": b1, "L1": L1, "lb1": lb1,
        "W2": W2, "as2": as2, "ad2": ad2, "b2": b2, "L2": L2, "lb2": lb2,
        "W3": W3, "as3": as3, "ad3": ad3, "b3": b3, "L3": L3, "lb3": lb3,
        "W4": W4, "as4": as4, "ad4": ad4, "b4": b4, "L4": L4, "lb4": lb4,
        "W5": W5, "as5": as5, "ad5": ad5, "b5": b5, "L5": L5, "lb5": lb5,
        "W6": W6, "as6": as6, "ad6": ad6, "b6": b6, "L6": L6, "lb6": lb6,
        "W7": W7, "as7": as7, "ad7": ad7, "b7": b7,
    }
    h = x
    for li in range(1, 8):
        W = p[f"W{li}"]
        dout = W.shape[1]
        P = _pad_p(dout)
        H = P // 2
        Wp = jnp.pad(W, ((0, 0), (0, P - dout)))
        a_sp = jnp.pad(p[f"as{li}"], (0, P - dout)).reshape(P, 1)
        a_dp = jnp.pad(p[f"ad{li}"], (0, P - dout)).reshape(P, 1)
        has_l = li < 7
        L = p[f"L{li}"] if has_l else None
        hW, aa, r, cvec, c11 = _dense_layer(h, Wp, a_sp, a_dp, L)
        h2 = jnp.stack([hW[:, :H], hW[:, H:]])
        blk = 64 if H > 64 else 128
        nblk = EDGE_SLOTS // blk
        cols = EDGE_SLOTS + blk
        src3 = src2[:, :cols].reshape(N_SUB, nblk + 1, blk)
        dst3 = dst2[:, :cols].reshape(N_SUB, nblk + 1, blk)
        o2 = _sc_edge(h2, aa[:, 0], aa[:, 1], cvec, src3, dst3,
                      H=H, dout=dout, BLK=blk)
        b2 = p[f"b{li}"].reshape(1, dout)
        if has_l:
            h = _norm_layer(o2[0], o2[1], hW, aa, c11, r, b2,
                            p[f"lb{li}"].reshape(1, dout), dout)
        else:
            return _norm_layer(o2[0], o2[1], hW, aa, c11, None, b2, None,
                               dout)


# speedup vs baseline: 1.4398x; 1.0211x over previous
import functools
import jax
import jax.numpy as jnp
from jax import lax
from jax.experimental import pallas as pl
from jax.experimental.pallas import tpu as pltpu
from jax.experimental.pallas import tpu_sc as plsc

N_NODES = 10000
N_SUB = 16
EDGE_VALID = 10000
EDGE_SLOTS = 10240
N_ACC = 10240
ROWS_PER_SUB = N_ACC // 16


def _lrelu(x, slope):
    return jnp.where(x >= 0, x, slope * x)


def _pad_p(dout):
    h = ((dout // 2) // 16 + 1) * 16
    return 2 * h


def _dense_body(x_ref, w_ref, as_ref, ad_ref, l_ref, h_ref, aa_ref, r_ref,
                cv_ref, c11_ref):
    x = x_ref[...]
    h = jnp.dot(x, w_ref[...], preferred_element_type=jnp.float32)
    h_ref[...] = h
    asrc = jnp.dot(h, as_ref[...], preferred_element_type=jnp.float32)
    adst = jnp.dot(h, ad_ref[...], preferred_element_type=jnp.float32)
    aa_ref[...] = jnp.concatenate([asrc, adst], axis=1)
    if l_ref is not None:
        r_ref[...] = jnp.dot(x, l_ref[...], preferred_element_type=jnp.float32)
    c = jnp.max(asrc) + jnp.max(adst)
    c = jnp.where(c >= 0, c, 0.2 * c)
    cv_ref[...] = jnp.full((16,), c, jnp.float32)
    c11_ref[...] = c.reshape(1, 1)


def _dense_layer(x, Wp, a_sp, a_dp, L):
    N = x.shape[0]
    P = Wp.shape[1]
    has_l = L is not None
    outs = [
        jax.ShapeDtypeStruct((N, P), jnp.float32),
        jax.ShapeDtypeStruct((N, 2), jnp.float32),
        jax.ShapeDtypeStruct((N, L.shape[1] if has_l else 1), jnp.float32),
        jax.ShapeDtypeStruct((16,), jnp.float32),
        jax.ShapeDtypeStruct((1, 1), jnp.float32),
    ]
    body = _dense_body if has_l else (
        lambda x_ref, w_ref, as_ref, ad_ref, h_ref, aa_ref, r_ref, cv_ref,
        c11_ref: _dense_body(x_ref, w_ref, as_ref, ad_ref, None, h_ref,
                             aa_ref, r_ref, cv_ref, c11_ref))
    args = (x, Wp, a_sp, a_dp) + ((L,) if has_l else ())
    return pl.pallas_call(body, out_shape=outs)(*args)


@functools.partial(jax.jit, static_argnames=("H", "dout", "BLK"))
def _sc_edge(h2, asrc, adst, cvec, src3, dst3, *, H, dout, BLK):
    mesh = plsc.VectorSubcoreMesh(core_axis_name="c", subcore_axis_name="s",
                                  num_cores=2)
    nvec = H // 16
    NBLK = EDGE_SLOTS // BLK

    @functools.partial(
        pl.kernel,
        out_type=jax.ShapeDtypeStruct((2, N_ACC, H), jnp.float32),
        mesh=mesh,
        scratch_types=[
            pltpu.VMEM((N_NODES,), jnp.float32),
            pltpu.VMEM((N_NODES,), jnp.float32),
            pltpu.VMEM((16,), jnp.float32),
            pltpu.VMEM((NBLK + 1, BLK), jnp.int32),
            pltpu.VMEM((NBLK + 1, BLK), jnp.int32),
            pltpu.VMEM((BLK,), jnp.float32),
            pltpu.VMEM((BLK,), jnp.float32),
            pltpu.VMEM((BLK, H), jnp.float32),
            pltpu.VMEM((BLK, H), jnp.float32),
            pltpu.VMEM_SHARED((N_ACC, H), jnp.float32),
            pltpu.SemaphoreType.DMA,
            pltpu.SemaphoreType.DMA,
        ],
        compiler_params=pltpu.CompilerParams(needs_layout_passes=False,
                                             use_tc_tiling_on_sc=False),
    )
    def k(h_hbm, asrc_hbm, adst_hbm, c_hbm, src_hbm, dst_hbm, out_hbm,
          asrc_v, adst_v, cv_v, src_v, dst_v, ex_a, ex_b, rows_a, rows_b,
          acc_sh, sem0, sem1):
        c = lax.axis_index("c")
        s = lax.axis_index("s")
        pltpu.sync_copy(asrc_hbm, asrc_v)
        pltpu.sync_copy(adst_hbm, adst_v)
        pltpu.sync_copy(c_hbm, cv_v)
        pltpu.sync_copy(src_hbm.at[s], src_v)
        pltpu.sync_copy(dst_hbm.at[s], dst_v)

        def zbody(i, _):
            for j in range(nvec):
                rows_a[i, pl.ds(j * 16, 16)] = jnp.zeros((16,), jnp.float32)
            return 0
        lax.fori_loop(0, BLK, zbody, 0)
        base = s * ROWS_PER_SUB
        for t in range(ROWS_PER_SUB // BLK):
            pltpu.sync_copy(rows_a, acc_sh.at[pl.ds(base + t * BLK, BLK)])
        plsc.subcore_barrier()

        cv = cv_v[...]
        colbase = c * H
        dmasks = [(colbase + j * 16 + lax.iota(jnp.int32, 16)) == dout
                  for j in range(nvec)]

        def compute_ex(b, ex_v):
            for j in range(BLK // 16):
                s16 = src_v[b, pl.ds(j * 16, 16)]
                d16 = dst_v[b, pl.ds(j * 16, 16)]
                a1 = plsc.load_gather(asrc_v, [s16])
                a2 = plsc.load_gather(adst_v, [d16])
                e = a1 + a2
                e = jnp.where(e >= 0, e, 0.2 * e) - cv
                ex = jnp.exp(e)
                gid = b * BLK + j * 16 + lax.iota(jnp.int32, 16)
                ex = jnp.where(gid < EDGE_VALID, ex, 0.0)
                ex_v[pl.ds(j * 16, 16)] = ex

        def start_gather(b, rows_v, sem):
            pltpu.async_copy(h_hbm.at[c].at[src_v.at[b]], rows_v, sem)

        def wait_gather(b, rows_v, sem):
            pltpu.make_async_copy(h_hbm.at[c].at[src_v.at[b]], rows_v,
                                  sem).wait()

        def process(b, ex_v, rows_v, sem, ex_n, rows_n, sem_n):
            compute_ex(b + 1, ex_n)
            start_gather(b + 1, rows_n, sem_n)
            wait_gather(b, rows_v, sem)

            def sbody(i2, _):
                for u in range(2):
                    i = 2 * i2 + u
                    exb = plsc.load_gather(
                        ex_v, [jnp.full((16,), i, jnp.int32)])
                    for j in range(nvec):
                        v = rows_v[i, pl.ds(j * 16, 16)] * exb
                        v = jnp.where(dmasks[j], exb, v)
                        rows_v[i, pl.ds(j * 16, 16)] = v
                return 0
            lax.fori_loop(0, BLK // 2, sbody, 0)
            pltpu.sync_copy(rows_v, acc_sh.at[dst_v.at[b]], add=True)

        compute_ex(0, ex_a)
        start_gather(0, rows_a, sem0)

        def pair(k2, _):
            process(2 * k2, ex_a, rows_a, sem0, ex_b, rows_b, sem1)
            process(2 * k2 + 1, ex_b, rows_b, sem1, ex_a, rows_a, sem0)
            return 0
        lax.fori_loop(0, NBLK // 2, pair, 0)
        wait_gather(NBLK, rows_a, sem0)

        plsc.subcore_barrier()
        for t in range(ROWS_PER_SUB // BLK):
            pltpu.sync_copy(acc_sh.at[pl.ds(base + t * BLK, BLK)],
                            out_hbm.at[c, pl.ds(base + t * BLK, BLK)])

    return k(h2, asrc, adst, cvec, src3, dst3)


def _norm_layer(o0, o1, h, aa, c11, r, b2, lb2, dout):
    N, P = h.shape
    H = P // 2
    has_l = r is not None
    RB = 2000
    grid = (N // RB,)

    def body(*refs):
        if has_l:
            o0_ref, o1_ref, h_ref, aa_ref, c_ref, r_ref, b_ref, lb_ref, out_ref = refs
        else:
            o0_ref, o1_ref, h_ref, aa_ref, c_ref, b_ref, out_ref = refs
        es = jnp.exp(_lrelu(aa_ref[:, 0:1] + aa_ref[:, 1:2], 0.2) - c_ref[0, 0])
        acc = jnp.concatenate([o0_ref[...], o1_ref[...]], axis=1)
        acc = acc + es * h_ref[...]
        denom = acc[:, dout:dout + 1] + es
        g = acc[:, 0:dout] / denom + b_ref[...]
        if has_l:
            out_ref[...] = _lrelu(g, 0.01) + r_ref[...] + lb_ref[...]
        else:
            out_ref[...] = g

    rowspec = lambda w: pl.BlockSpec((RB, w), lambda i: (i, 0))
    fixspec = lambda a, b: pl.BlockSpec((a, b), lambda i: (0, 0))
    in_specs = [rowspec(H), rowspec(H), rowspec(P), rowspec(2), fixspec(1, 1)]
    args = [o0, o1, h, aa, c11]
    if has_l:
        in_specs += [rowspec(r.shape[1]), fixspec(1, dout), fixspec(1, dout)]
        args += [r, b2, lb2]
    else:
        in_specs += [fixspec(1, dout)]
        args += [b2]
    return pl.pallas_call(
        body, grid=grid, in_specs=in_specs,
        out_specs=rowspec(dout),
        out_shape=jax.ShapeDtypeStruct((N, dout), jnp.float32))(*args)


def kernel(x, edge_index,
           W1, as1, ad1, b1, L1, lb1,
           W2, as2, ad2, b2, L2, lb2,
           W3, as3, ad3, b3, L3, lb3,
           W4, as4, ad4, b4, L4, lb4,
           W5, as5, ad5, b5, L5, lb5,
           W6, as6, ad6, b6, L6, lb6,
           W7, as7, ad7, b7):
    src = edge_index[0].astype(jnp.int32)
    dst = edge_index[1].astype(jnp.int32)
    padw = EDGE_SLOTS - EDGE_VALID + 256
    padv = (jnp.arange(N_SUB * padw, dtype=jnp.int32) % N_NODES).reshape(
        N_SUB, padw)
    src2 = jnp.concatenate([src.reshape(N_SUB, EDGE_VALID), padv], axis=1)
    dst2 = jnp.concatenate([dst.reshape(N_SUB, EDGE_VALID), padv], axis=1)

    p = {
        "W1": W1, "as1": as1, "ad1": ad1, "b1": b1, "L1": L1, "lb1": lb1,
        "W2": W2, "as2": as2, "ad2": ad2, "b2": b2, "L2": L2, "lb2": lb2,
        "W3": W3, "as3": as3, "ad3": ad3, "b3": b3, "L3": L3, "lb3": lb3,
        "W4": W4, "as4": as4, "ad4": ad4, "b4": b4, "L4": L4, "lb4": lb4,
        "W5": W5, "as5": as5, "ad5": ad5, "b5": b5, "L5": L5, "lb5": lb5,
        "W6": W6, "as6": as6, "ad6": ad6, "b6": b6, "L6": L6, "lb6": lb6,
        "W7": W7, "as7": as7, "ad7": ad7, "b7": b7,
    }
    h = x
    for li in range(1, 8):
        W = p[f"W{li}"]
        dout = W.shape[1]
        P = _pad_p(dout)
        H = P // 2
        Wp = jnp.pad(W, ((0, 0), (0, P - dout)))
        a_sp = jnp.pad(p[f"as{li}"], (0, P - dout)).reshape(P, 1)
        a_dp = jnp.pad(p[f"ad{li}"], (0, P - dout)).reshape(P, 1)
        has_l = li < 7
        L = p[f"L{li}"] if has_l else None
        hW, aa, r, cvec, c11 = _dense_layer(h, Wp, a_sp, a_dp, L)
        h2 = jnp.stack([hW[:, :H], hW[:, H:]])
        blk = 64 if H > 64 else 128
        nblk = EDGE_SLOTS // blk
        cols = EDGE_SLOTS + blk
        src3 = src2[:, :cols].reshape(N_SUB, nblk + 1, blk)
        dst3 = dst2[:, :cols].reshape(N_SUB, nblk + 1, blk)
        o2 = _sc_edge(h2, aa[:, 0], aa[:, 1], cvec, src3, dst3,
                      H=H, dout=dout, BLK=blk)
        b2 = p[f"b{li}"].reshape(1, dout)
        if has_l:
            h = _norm_layer(o2[0], o2[1], hW, aa, c11, r, b2,
                            p[f"lb{li}"].reshape(1, dout), dout)
        else:
            return _norm_layer(o2[0], o2[1], hW, aa, c11, None, b2, None,
                               dout)
